# baseline (speedup 1.0000x reference)
import functools

import jax
import jax.numpy as jnp
from jax import lax
from jax.experimental import pallas as pl
from jax.experimental.pallas import tpu as pltpu
from jax.experimental.pallas import tpu_sc as plsc

_B = 64
_S = 576
_D = 768
_SB = 8
_NBLK = (_S + 1 + _SB - 1) // _SB
_NBUF = 3


def _in_dmas(in_hbm, buf, sem, j):
    for s in range(_SB):
        row = jnp.clip(_SB * j + s - 1, 0, _S - 1)
        pltpu.async_copy(in_hbm.at[:, row, :], buf.at[s], sem)


def _wait_in(in_hbm, buf, sem):
    for s in range(_SB):
        pltpu.make_async_copy(in_hbm.at[:, 0, :], buf.at[s], sem).wait()


def _tc_body(in_hbm, pos_ref, cls_ref, out_ref, b0, b1, b2, in_sems):
    j = pl.program_id(0)
    bufs = (b0, b1, b2)

    @pl.when(j == 0)
    def _():
        _in_dmas(in_hbm, bufs[0], in_sems.at[0], 0)
        _in_dmas(in_hbm, bufs[1], in_sems.at[1], 1)

    for p in range(_NBUF):
        @pl.when(j % _NBUF == p)
        def _(p=p):
            _wait_in(in_hbm, bufs[p], in_sems.at[p])

            pos_b = jnp.broadcast_to(
                pos_ref[...].reshape(_SB, 1, _D), (_SB, _B, _D))
            out_ref[...] = bufs[p][...] + pos_b

            @pl.when(j + 2 < _NBLK)
            def _():
                q = (p + 2) % _NBUF
                _in_dmas(in_hbm, bufs[q], in_sems.at[q], j + 2)

    @pl.when(j == 0)
    def _():
        out_ref[0:1, :, :] = cls_ref[...].reshape(1, _B, _D)


@jax.jit
def _run_tc(inputs, position_table, cls_row):
    pos_pad = lax.slice(position_table, (0, 0), (_NBLK * _SB, _D))
    out_t = pl.pallas_call(
        _tc_body,
        grid=(_NBLK,),
        in_specs=[
            pl.BlockSpec(memory_space=pltpu.HBM),
            pl.BlockSpec((_SB, _D), lambda j: (j, 0)),
            pl.BlockSpec((_B, _D), lambda j: (0, 0)),
        ],
        out_specs=pl.BlockSpec((_SB, _B, _D), lambda j: (j, 0, 0)),
        out_shape=jax.ShapeDtypeStruct((_S + 1, _B, _D), jnp.float32),
        scratch_shapes=[
            pltpu.VMEM((_SB, _B, _D), jnp.float32),
            pltpu.VMEM((_SB, _B, _D), jnp.float32),
            pltpu.VMEM((_SB, _B, _D), jnp.float32),
            pltpu.SemaphoreType.DMA((_NBUF,)),
        ],
    )(inputs, pos_pad, cls_row)
    return out_t


_LANES = 16
_CLS_CHUNK = _D // _LANES


def _sc_body(cls_h, pos_h, out_h, cls_v, p0_v):
    wid = lax.axis_index("s") * 2 + lax.axis_index("c")

    pltpu.sync_copy(cls_h, cls_v)
    pltpu.sync_copy(pos_h.at[pl.ds(0, _D)], p0_v)

    @plsc.parallel_loop(0, _CLS_CHUNK, 1, unroll=8)
    def _(i):
        sl = pl.ds(i * _LANES, _LANES)
        plsc.addupdate(cls_v.at[sl], p0_v[sl])

    pltpu.sync_copy(cls_v, out_h.at[pl.ds((2 * wid) * _D, _D)])
    pltpu.sync_copy(cls_v, out_h.at[pl.ds((2 * wid + 1) * _D, _D)])


@jax.jit
def _run_sc(cls_flat, pos_flat):
    mesh = plsc.VectorSubcoreMesh(core_axis_name="c", subcore_axis_name="s")
    f = pl.kernel(
        _sc_body,
        out_type=jax.ShapeDtypeStruct((_B * _D,), jnp.float32),
        mesh=mesh,
        scratch_types=[
            pltpu.VMEM((_D,), jnp.float32),
            pltpu.VMEM((_D,), jnp.float32),
        ],
    )
    return f(cls_flat, pos_flat)


def kernel(inputs, class_embed, position_table):
    sc_row = _run_sc(class_embed.reshape(_D), position_table[0].reshape(_D))
    out_t = _run_tc(inputs, position_table, sc_row.reshape(_B, _D))
    return out_t.transpose(1, 0, 2)


# speedup vs baseline: 1.0153x; 1.0153x over previous
import functools

import jax
import jax.numpy as jnp
from jax import lax
from jax.experimental import pallas as pl
from jax.experimental.pallas import tpu as pltpu
from jax.experimental.pallas import tpu_sc as plsc

_B = 64
_S = 576
_D = 768
_SB = 8
_NBLK = (_S + 1 + _SB - 1) // _SB
_NBUF = 3


def _in_dmas(in_hbm, buf, sem, j):
    for s in range(_SB):
        row = jnp.clip(_SB * j + s - 1, 0, _S - 1)
        pltpu.async_copy(in_hbm.at[:, row, :], buf.at[s], sem)


def _wait_in(in_hbm, buf, sem):
    for s in range(_SB):
        pltpu.make_async_copy(in_hbm.at[:, 0, :], buf.at[s], sem).wait()


def _tc_body(in_hbm, pos_ref, out_ref, b0, b1, b2, in_sems):
    j = pl.program_id(0)
    bufs = (b0, b1, b2)

    @pl.when(j == 0)
    def _():
        _in_dmas(in_hbm, bufs[0], in_sems.at[0], 0)
        _in_dmas(in_hbm, bufs[1], in_sems.at[1], 1)

    for p in range(_NBUF):
        @pl.when(j % _NBUF == p)
        def _(p=p):
            _wait_in(in_hbm, bufs[p], in_sems.at[p])

            pos_b = jnp.broadcast_to(
                pos_ref[...].reshape(_SB, 1, _D), (_SB, _B, _D))
            out_ref[...] = bufs[p][...] + pos_b

            @pl.when(j + 2 < _NBLK)
            def _():
                q = (p + 2) % _NBUF
                _in_dmas(in_hbm, bufs[q], in_sems.at[q], j + 2)


@jax.jit
def _run_tc(inputs, position_table):
    pos_pad = lax.slice(position_table, (0, 0), (_NBLK * _SB, _D))
    out_t = pl.pallas_call(
        _tc_body,
        grid=(_NBLK,),
        in_specs=[
            pl.BlockSpec(memory_space=pltpu.HBM),
            pl.BlockSpec((_SB, _D), lambda j: (j, 0)),
        ],
        out_specs=pl.BlockSpec((_SB, _B, _D), lambda j: (j, 0, 0)),
        out_shape=jax.ShapeDtypeStruct((_S + 1, _B, _D), jnp.float32),
        scratch_shapes=[
            pltpu.VMEM((_SB, _B, _D), jnp.float32),
            pltpu.VMEM((_SB, _B, _D), jnp.float32),
            pltpu.VMEM((_SB, _B, _D), jnp.float32),
            pltpu.SemaphoreType.DMA((_NBUF,)),
        ],
    )(inputs, pos_pad)
    return out_t


_LANES = 16
_CLS_CHUNK = _D // _LANES


def _sc_body(cls_h, pos_h, out_h, cls_v, p0_v):
    wid = lax.axis_index("s") * 2 + lax.axis_index("c")

    pltpu.sync_copy(cls_h, cls_v)
    pltpu.sync_copy(pos_h.at[pl.ds(0, _D)], p0_v)

    @plsc.parallel_loop(0, _CLS_CHUNK, 1, unroll=8)
    def _(i):
        sl = pl.ds(i * _LANES, _LANES)
        plsc.addupdate(cls_v.at[sl], p0_v[sl])

    pltpu.sync_copy(cls_v, out_h.at[pl.ds((2 * wid) * _D, _D)])
    pltpu.sync_copy(cls_v, out_h.at[pl.ds((2 * wid + 1) * _D, _D)])


@jax.jit
def _run_sc(cls_flat, pos_flat):
    mesh = plsc.VectorSubcoreMesh(core_axis_name="c", subcore_axis_name="s")
    f = pl.kernel(
        _sc_body,
        out_type=jax.ShapeDtypeStruct((_B * _D,), jnp.float32),
        mesh=mesh,
        scratch_types=[
            pltpu.VMEM((_D,), jnp.float32),
            pltpu.VMEM((_D,), jnp.float32),
        ],
    )
    return f(cls_flat, pos_flat)


def kernel(inputs, class_embed, position_table):
    pos0 = lax.slice(position_table, (0, 0), (1, _D)).reshape(_D)
    sc_row = _run_sc(class_embed.reshape(_D), pos0)
    out_t = _run_tc(inputs, position_table)
    out_t = lax.dynamic_update_slice(
        out_t, sc_row.reshape(1, _B, _D), (0, 0, 0))
    return out_t.transpose(1, 0, 2)


# speedup vs baseline: 1.0162x; 1.0008x over previous
import jax
import jax.numpy as jnp
from jax import lax
from jax.experimental import pallas as pl
from jax.experimental.pallas import tpu as pltpu
from jax.experimental.pallas import tpu_sc as plsc

_B = 64
_S = 576
_D = 768
_SB = 8
_NBLK = (_S + 1 + _SB - 1) // _SB
_NBUF = 3


def _in_dmas(in_hbm, buf, sem, j):
    for s in range(_SB):
        row = jnp.clip(_SB * j + s - 1, 0, _S - 1)
        pltpu.async_copy(in_hbm.at[:, row, :], buf.at[s], sem)


def _wait_in(in_hbm, buf, sem):
    for s in range(_SB):
        pltpu.make_async_copy(in_hbm.at[:, 0, :], buf.at[s], sem).wait()


def _tc_body(in_hbm, pos_ref, out_ref, b0, b1, b2, in_sems):
    j = pl.program_id(0)
    bufs = (b0, b1, b2)

    @pl.when(j == 0)
    def _():
        _in_dmas(in_hbm, bufs[0], in_sems.at[0], 0)
        _in_dmas(in_hbm, bufs[1], in_sems.at[1], 1)

    for p in range(_NBUF):
        @pl.when(j % _NBUF == p)
        def _(p=p):
            _wait_in(in_hbm, bufs[p], in_sems.at[p])

            pos_b = jnp.broadcast_to(
                pos_ref[...].reshape(_SB, 1, _D), (_SB, _B, _D))
            out_ref[...] = bufs[p][...] + pos_b

            @pl.when(j + 2 < _NBLK)
            def _():
                q = (p + 2) % _NBUF
                _in_dmas(in_hbm, bufs[q], in_sems.at[q], j + 2)


@jax.jit
def _run_tc(inputs, position_table):
    pos_pad = lax.slice(position_table, (0, 0), (_NBLK * _SB, _D))
    out_t = pl.pallas_call(
        _tc_body,
        grid=(_NBLK,),
        in_specs=[
            pl.BlockSpec(memory_space=pltpu.HBM),
            pl.BlockSpec((_SB, _D), lambda j: (j, 0)),
        ],
        out_specs=pl.BlockSpec((_SB, _B, _D), lambda j: (j, 0, 0)),
        out_shape=jax.ShapeDtypeStruct((_S + 1, _B, _D), jnp.float32),
        scratch_shapes=[
            pltpu.VMEM((_SB, _B, _D), jnp.float32),
            pltpu.VMEM((_SB, _B, _D), jnp.float32),
            pltpu.VMEM((_SB, _B, _D), jnp.float32),
            pltpu.SemaphoreType.DMA((_NBUF,)),
        ],
    )(inputs, pos_pad)
    return out_t


_LANES = 16
_CLS_CHUNK = _D // _LANES


def _sc_body(cls_h, pos_h, out_h, cls_v, p0_v):
    wid = lax.axis_index("s") * 2 + lax.axis_index("c")

    pltpu.sync_copy(cls_h, cls_v)
    pltpu.sync_copy(pos_h.at[pl.ds(0, _D)], p0_v)

    @plsc.parallel_loop(0, _CLS_CHUNK, 1, unroll=8)
    def _(i):
        sl = pl.ds(i * _LANES, _LANES)
        plsc.addupdate(cls_v.at[sl], p0_v[sl])

    pltpu.sync_copy(cls_v, out_h.at[pl.ds((2 * wid) * _D, _D)])
    pltpu.sync_copy(cls_v, out_h.at[pl.ds((2 * wid + 1) * _D, _D)])


@jax.jit
def _run_sc(cls_flat, pos_flat):
    mesh = plsc.VectorSubcoreMesh(core_axis_name="c", subcore_axis_name="s")
    f = pl.kernel(
        _sc_body,
        out_type=jax.ShapeDtypeStruct((_B * _D,), jnp.float32),
        mesh=mesh,
        scratch_types=[
            pltpu.VMEM((_D,), jnp.float32),
            pltpu.VMEM((_D,), jnp.float32),
        ],
    )
    return f(cls_flat, pos_flat)


def kernel(inputs, class_embed, position_table):
    pos0 = lax.slice(position_table, (0, 0), (1, _D)).reshape(_D)
    sc_row = _run_sc(class_embed.reshape(_D), pos0)
    out_t = _run_tc(inputs, position_table)
    out_t = lax.dynamic_update_slice(
        out_t, sc_row.reshape(1, _B, _D), (0, 0, 0))
    return out_t.transpose(1, 0, 2)


# speedup vs baseline: 1.0636x; 1.0466x over previous
import jax
import jax.numpy as jnp
from jax import lax
from jax.experimental import pallas as pl
from jax.experimental.pallas import tpu as pltpu
from jax.experimental.pallas import tpu_sc as plsc

_B = 64
_S = 576
_D = 768
_SB = 8
_NBLK = (_S + 1 + _SB - 1) // _SB
_NBUF = 4


def _in_dmas(in_hbm, buf, sem, j):
    for s in range(_SB):
        row = jnp.clip(_SB * j + s - 1, 0, _S - 1)
        pltpu.async_copy(in_hbm.at[:, row, :], buf.at[s], sem)


def _wait_in(in_hbm, buf, sem):
    for s in range(_SB):
        pltpu.make_async_copy(in_hbm.at[:, 0, :], buf.at[s], sem).wait()


def _tc_body(in_hbm, pos_ref, out_ref, b0, b1, b2, b3, in_sems):
    j = pl.program_id(0)
    bufs = (b0, b1, b2, b3)

    @pl.when(j == 0)
    def _():
        _in_dmas(in_hbm, bufs[0], in_sems.at[0], 0)
        _in_dmas(in_hbm, bufs[1], in_sems.at[1], 1)
        _in_dmas(in_hbm, bufs[2], in_sems.at[2], 2)

    for p in range(_NBUF):
        @pl.when(j % _NBUF == p)
        def _(p=p):
            _wait_in(in_hbm, bufs[p], in_sems.at[p])

            pos_b = jnp.broadcast_to(
                pos_ref[...].reshape(_SB, 1, _D), (_SB, _B, _D))
            out_ref[...] = bufs[p][...] + pos_b

            @pl.when(j + 3 < _NBLK)
            def _():
                q = (p + 3) % _NBUF
                _in_dmas(in_hbm, bufs[q], in_sems.at[q], j + 3)


@jax.jit
def _run_tc(inputs, position_table):
    pos_pad = lax.slice(position_table, (0, 0), (_NBLK * _SB, _D))
    out_t = pl.pallas_call(
        _tc_body,
        grid=(_NBLK,),
        in_specs=[
            pl.BlockSpec(memory_space=pltpu.HBM),
            pl.BlockSpec((_SB, _D), lambda j: (j, 0)),
        ],
        out_specs=pl.BlockSpec((_SB, _B, _D), lambda j: (j, 0, 0)),
        out_shape=jax.ShapeDtypeStruct((_S + 1, _B, _D), jnp.float32),
        scratch_shapes=[
            pltpu.VMEM((_SB, _B, _D), jnp.float32),
            pltpu.VMEM((_SB, _B, _D), jnp.float32),
            pltpu.VMEM((_SB, _B, _D), jnp.float32),
            pltpu.VMEM((_SB, _B, _D), jnp.float32),
            pltpu.SemaphoreType.DMA((_NBUF,)),
        ],
    )(inputs, pos_pad)
    return out_t


_LANES = 16
_CLS_CHUNK = _D // _LANES


def _sc_body(cls_h, pos_h, out_h, cls_v, p0_v):
    wid = lax.axis_index("s") * 2 + lax.axis_index("c")

    pltpu.sync_copy(cls_h, cls_v)
    pltpu.sync_copy(pos_h.at[pl.ds(0, _D)], p0_v)

    @plsc.parallel_loop(0, _CLS_CHUNK, 1, unroll=8)
    def _(i):
        sl = pl.ds(i * _LANES, _LANES)
        plsc.addupdate(cls_v.at[sl], p0_v[sl])

    pltpu.sync_copy(cls_v, out_h.at[pl.ds((2 * wid) * _D, _D)])
    pltpu.sync_copy(cls_v, out_h.at[pl.ds((2 * wid + 1) * _D, _D)])


@jax.jit
def _run_sc(cls_flat, pos_flat):
    mesh = plsc.VectorSubcoreMesh(core_axis_name="c", subcore_axis_name="s")
    f = pl.kernel(
        _sc_body,
        out_type=jax.ShapeDtypeStruct((_B * _D,), jnp.float32),
        mesh=mesh,
        scratch_types=[
            pltpu.VMEM((_D,), jnp.float32),
            pltpu.VMEM((_D,), jnp.float32),
        ],
    )
    return f(cls_flat, pos_flat)


def kernel(inputs, class_embed, position_table):
    pos0 = lax.slice(position_table, (0, 0), (1, _D)).reshape(_D)
    sc_row = _run_sc(class_embed.reshape(_D), pos0)
    out_t = _run_tc(inputs, position_table)
    out_t = lax.dynamic_update_slice(
        out_t, sc_row.reshape(1, _B, _D), (0, 0, 0))
    return out_t.transpose(1, 0, 2)


# speedup vs baseline: 1.0678x; 1.0039x over previous
import jax
import jax.numpy as jnp
from jax import lax
from jax.experimental import pallas as pl
from jax.experimental.pallas import tpu as pltpu
from jax.experimental.pallas import tpu_sc as plsc

_B = 64
_S = 576
_D = 768
_SB = 8
_NBLK = (_S + 1 + _SB - 1) // _SB
_NBUF = 6


def _in_dmas(in_hbm, buf, sem, j):
    for s in range(_SB):
        row = jnp.clip(_SB * j + s - 1, 0, _S - 1)
        pltpu.async_copy(in_hbm.at[:, row, :], buf.at[s], sem)


def _wait_in(in_hbm, buf, sem):
    for s in range(_SB):
        pltpu.make_async_copy(in_hbm.at[:, 0, :], buf.at[s], sem).wait()


def _tc_body(in_hbm, pos_ref, out_ref, b0, b1, b2, b3, b4, b5, in_sems):
    j = pl.program_id(0)
    bufs = (b0, b1, b2, b3, b4, b5)

    @pl.when(j == 0)
    def _():
        _in_dmas(in_hbm, bufs[0], in_sems.at[0], 0)
        _in_dmas(in_hbm, bufs[1], in_sems.at[1], 1)
        _in_dmas(in_hbm, bufs[2], in_sems.at[2], 2)
        _in_dmas(in_hbm, bufs[3], in_sems.at[3], 3)
        _in_dmas(in_hbm, bufs[4], in_sems.at[4], 4)

    for p in range(_NBUF):
        @pl.when(j % _NBUF == p)
        def _(p=p):
            _wait_in(in_hbm, bufs[p], in_sems.at[p])

            pos_b = jnp.broadcast_to(
                pos_ref[...].reshape(_SB, 1, _D), (_SB, _B, _D))
            out_ref[...] = bufs[p][...] + pos_b

            @pl.when(j + 5 < _NBLK)
            def _():
                q = (p + 5) % _NBUF
                _in_dmas(in_hbm, bufs[q], in_sems.at[q], j + 5)


@jax.jit
def _run_tc(inputs, position_table):
    pos_pad = lax.slice(position_table, (0, 0), (_NBLK * _SB, _D))
    out_t = pl.pallas_call(
        _tc_body,
        grid=(_NBLK,),
        in_specs=[
            pl.BlockSpec(memory_space=pltpu.HBM),
            pl.BlockSpec((_SB, _D), lambda j: (j, 0)),
        ],
        out_specs=pl.BlockSpec((_SB, _B, _D), lambda j: (j, 0, 0)),
        out_shape=jax.ShapeDtypeStruct((_S + 1, _B, _D), jnp.float32),
        scratch_shapes=[
            pltpu.VMEM((_SB, _B, _D), jnp.float32),
            pltpu.VMEM((_SB, _B, _D), jnp.float32),
            pltpu.VMEM((_SB, _B, _D), jnp.float32),
            pltpu.VMEM((_SB, _B, _D), jnp.float32),
            pltpu.VMEM((_SB, _B, _D), jnp.float32),
            pltpu.VMEM((_SB, _B, _D), jnp.float32),
            pltpu.SemaphoreType.DMA((_NBUF,)),
        ],
    )(inputs, pos_pad)
    return out_t


_LANES = 16
_CLS_CHUNK = _D // _LANES


def _sc_body(cls_h, pos_h, out_h, cls_v, p0_v):
    wid = lax.axis_index("s") * 2 + lax.axis_index("c")

    pltpu.sync_copy(cls_h, cls_v)
    pltpu.sync_copy(pos_h.at[pl.ds(0, _D)], p0_v)

    @plsc.parallel_loop(0, _CLS_CHUNK, 1, unroll=8)
    def _(i):
        sl = pl.ds(i * _LANES, _LANES)
        plsc.addupdate(cls_v.at[sl], p0_v[sl])

    pltpu.sync_copy(cls_v, out_h.at[pl.ds((2 * wid) * _D, _D)])
    pltpu.sync_copy(cls_v, out_h.at[pl.ds((2 * wid + 1) * _D, _D)])


@jax.jit
def _run_sc(cls_flat, pos_flat):
    mesh = plsc.VectorSubcoreMesh(core_axis_name="c", subcore_axis_name="s")
    f = pl.kernel(
        _sc_body,
        out_type=jax.ShapeDtypeStruct((_B * _D,), jnp.float32),
        mesh=mesh,
        scratch_types=[
            pltpu.VMEM((_D,), jnp.float32),
            pltpu.VMEM((_D,), jnp.float32),
        ],
    )
    return f(cls_flat, pos_flat)


def kernel(inputs, class_embed, position_table):
    pos0 = lax.slice(position_table, (0, 0), (1, _D)).reshape(_D)
    sc_row = _run_sc(class_embed.reshape(_D), pos0)
    out_t = _run_tc(inputs, position_table)
    out_t = lax.dynamic_update_slice(
        out_t, sc_row.reshape(1, _B, _D), (0, 0, 0))
    return out_t.transpose(1, 0, 2)
